# baseline (speedup 1.0000x reference)
import jax
import jax.numpy as jnp
from jax import lax
from jax.experimental import pallas as pl
from jax.experimental.pallas import tpu as pltpu
from jax.experimental.pallas import tpu_sc as plsc

_N = 1000
_F = 16
_B = 1024
_ROW = _N * _F

_info = plsc.get_sparse_core_info()
_NC = _info.num_cores
_NS = _info.num_subcores
_NW = _NC * _NS
_BPW = _B // _NW


_REP = 8
_NDMA = _BPW // _REP


def _broadcast_body(table_hbm, out_hbm, buf, sem_in, sem_out):
    wid = lax.axis_index("s") * _NC + lax.axis_index("c")
    base = wid * _BPW * _ROW
    fills = [
        pltpu.make_async_copy(table_hbm, buf.at[pl.ds(j * _ROW, _ROW)], sem_in)
        for j in range(_REP)
    ]
    for c in fills:
        c.start()
    for c in fills:
        c.wait()
    copies = [
        pltpu.make_async_copy(
            buf, out_hbm.at[pl.ds(base + i * _REP * _ROW, _REP * _ROW)], sem_out
        )
        for i in range(_NDMA)
    ]
    for c in copies:
        c.start()
    for c in copies:
        c.wait()


@jax.jit
def kernel(x, embed_weight):
    del x
    table = embed_weight.reshape(_ROW)
    mesh = plsc.VectorSubcoreMesh(core_axis_name="c", subcore_axis_name="s")
    out = pl.kernel(
        _broadcast_body,
        out_type=jax.ShapeDtypeStruct((_B * _ROW,), jnp.float32),
        mesh=mesh,
        scratch_types=[
            pltpu.VMEM((_REP * _ROW,), jnp.float32),
            pltpu.SemaphoreType.DMA,
            pltpu.SemaphoreType.DMA,
        ],
    )(table)
    return out.reshape(_B, _N, _F)


# speedup vs baseline: 1.3954x; 1.3954x over previous
import jax
import jax.numpy as jnp
from jax.experimental import pallas as pl
from jax.experimental.pallas import tpu as pltpu

_N = 1000
_F = 16
_B = 1024
_REP = 32
_NDMA = _B // _REP


def _broadcast_body(table_ref, out_hbm, buf, sem):
    buf[...] = jnp.broadcast_to(table_ref[...][None], (_REP, _N, _F))
    copies = [
        pltpu.make_async_copy(buf, out_hbm.at[pl.ds(i * _REP, _REP)], sem)
        for i in range(_NDMA)
    ]
    for c in copies:
        c.start()
    for c in copies:
        c.wait()


@jax.jit
def kernel(x, embed_weight):
    del x
    out = pl.pallas_call(
        _broadcast_body,
        out_shape=jax.ShapeDtypeStruct((_B, _N, _F), jnp.float32),
        in_specs=[pl.BlockSpec(memory_space=pltpu.VMEM)],
        out_specs=pl.BlockSpec(memory_space=pl.ANY),
        scratch_shapes=[
            pltpu.VMEM((_REP, _N, _F), jnp.float32),
            pltpu.SemaphoreType.DMA,
        ],
    )(embed_weight)
    return out


# speedup vs baseline: 24.2731x; 17.3946x over previous
import jax
import jax.numpy as jnp
from jax.experimental import pallas as pl
from jax.experimental.pallas import tpu as pltpu

_N = 1000
_F = 16
_B = 1024
_NB = 200
_G = _N // _NB


def _broadcast_body(w_ref, out_ref):
    out_ref[...] = jnp.broadcast_to(w_ref[...][:, :, None], (_NB, _F, _B))


@jax.jit
def kernel(x, embed_weight):
    del x
    out_t = pl.pallas_call(
        _broadcast_body,
        grid=(_G,),
        in_specs=[pl.BlockSpec((_NB, _F), lambda i: (i, 0))],
        out_specs=pl.BlockSpec((_NB, _F, _B), lambda i: (i, 0, 0)),
        out_shape=jax.ShapeDtypeStruct((_N, _F, _B), jnp.float32),
    )(embed_weight)
    return jnp.transpose(out_t, (2, 0, 1))


# speedup vs baseline: 26.0999x; 1.0753x over previous
import jax
import jax.numpy as jnp
from jax.experimental import pallas as pl
from jax.experimental.pallas import tpu as pltpu

_N = 1000
_F = 16
_B = 1024
_NB = 200
_G = _N // _NB


def _broadcast_body(w_ref, out_ref, wt_ref):
    i = pl.program_id(0)

    @pl.when(i == 0)
    def _():
        wt_ref[...] = w_ref[...].T

    chunk = wt_ref[pl.ds(i * _NB, _NB), :]
    out_ref[...] = jnp.broadcast_to(chunk[:, :, None], (_NB, _F, _B))


@jax.jit
def kernel(x, embed_weight):
    del x
    out_t = pl.pallas_call(
        _broadcast_body,
        grid=(_G,),
        in_specs=[pl.BlockSpec((_F, _N), lambda i: (0, 0))],
        out_specs=pl.BlockSpec((_NB, _F, _B), lambda i: (i, 0, 0)),
        out_shape=jax.ShapeDtypeStruct((_N, _F, _B), jnp.float32),
        scratch_shapes=[pltpu.VMEM((_N, _F), jnp.float32)],
    )(embed_weight.T)
    return jnp.transpose(out_t, (2, 0, 1))
